# baseline (speedup 1.0000x reference)
import functools

import jax
import jax.numpy as jnp
from jax import lax
from jax.experimental import pallas as pl
from jax.experimental.pallas import tpu as pltpu
from jax.experimental.pallas import tpu_sc as plsc

_B = 16384
_ED = 32
_PACK = 8
_NC = 2
_NS = 16
_NW = _NC * _NS
_BPW = _B // _NW
_CH = 128
_NCH = _BPW // _CH

_NUM_GENDERS = 2
_NUM_OCC = 21
_NUM_GENRES = 18
_H = 128

_TB = 2048


def _sc_gather_body(uidx_hbm, midx_hbm, utab_hbm, mtab_hbm,
                    uemb_hbm, memb_hbm,
                    uidx_v, midx_v, urows_v, mrows_v, sem_u, sem_m):
    wid = lax.axis_index("s") * _NC + lax.axis_index("c")
    base = wid * _NCH
    pltpu.sync_copy(uidx_hbm.at[pl.ds(base, _NCH)], uidx_v)
    pltpu.sync_copy(midx_hbm.at[pl.ds(base, _NCH)], midx_v)
    for r in range(_NCH):
        for i in range(_CH // 16):
            s = pl.ds(i * 16, 16)
            uidx_v[r, s] = uidx_v[r, s] >> 3
            midx_v[r, s] = midx_v[r, s] >> 3
    for c in range(_NCH):
        cu = pltpu.async_copy(utab_hbm.at[uidx_v.at[c]], urows_v, sem_u)
        cm = pltpu.async_copy(mtab_hbm.at[midx_v.at[c]], mrows_v, sem_m)
        cu.wait()
        cm.wait()
        row0 = (base + c) * _CH
        pltpu.sync_copy(urows_v, uemb_hbm.at[pl.ds(row0, _CH)])
        pltpu.sync_copy(mrows_v, memb_hbm.at[pl.ds(row0, _CH)])


@functools.cache
def _sc_gather():
    return pl.kernel(
        _sc_gather_body,
        out_type=(jax.ShapeDtypeStruct((_B, 128), jnp.int32),
                  jax.ShapeDtypeStruct((_B, 128), jnp.int32)),
        mesh=plsc.VectorSubcoreMesh(core_axis_name="c", subcore_axis_name="s",
                                    num_cores=_NC, num_subcores=_NS),
        scratch_types=[
            pltpu.VMEM((_NCH, _CH), jnp.int32),
            pltpu.VMEM((_NCH, _CH), jnp.int32),
            pltpu.VMEM((_CH, 128), jnp.int32),
            pltpu.VMEM((_CH, 128), jnp.int32),
            pltpu.SemaphoreType.DMA,
            pltpu.SemaphoreType.DMA,
        ],
    )


def _expand_halves(words, f32):
    lo = lax.bitcast_convert_type(
        jnp.left_shift(jnp.bitwise_and(words, 0xFFFF), 16), f32)
    hi = lax.bitcast_convert_type(
        jnp.bitwise_and(words, jnp.int32(-65536)), f32)
    return lo, hi


def _mlp_body(upack, mpack, user, movie, gender, occ, genres, gtab, otab,
              wg, bg, w1u_lo, w1u_hi, w1m_lo, w1m_hi, w1, b1, w2, b2, out):
    w1r = w1[...]
    f32 = jnp.float32
    genre_w = jnp.dot(wg[...], w1r[128:160, :], preferred_element_type=f32)
    gt_w = jnp.dot(gtab[...], w1r[64:96, :], preferred_element_type=f32)
    ot_w = jnp.dot(otab[...], w1r[96:128, :], preferred_element_type=f32)
    bias = b1[...] + jnp.dot(bg[...], w1r[128:160, :], preferred_element_type=f32)

    word_sub = lax.broadcasted_iota(jnp.int32, (_TB, 128), 1) >> 4
    umask = word_sub == (user[...] & 7)
    mmask = word_sub == (movie[...] & 7)
    uw = jnp.where(umask, upack[...], 0)
    mw = jnp.where(mmask, mpack[...], 0)
    ulo, uhi = _expand_halves(uw, f32)
    mlo, mhi = _expand_halves(mw, f32)

    g1h = (lax.broadcasted_iota(jnp.int32, (_TB, _NUM_GENDERS), 1)
           == gender[...]).astype(f32)
    o1h = (lax.broadcasted_iota(jnp.int32, (_TB, _NUM_OCC), 1)
           == occ[...]).astype(f32)

    h = (bias
         + jnp.dot(ulo, w1u_lo[...], preferred_element_type=f32)
         + jnp.dot(uhi, w1u_hi[...], preferred_element_type=f32)
         + jnp.dot(mlo, w1m_lo[...], preferred_element_type=f32)
         + jnp.dot(mhi, w1m_hi[...], preferred_element_type=f32)
         + jnp.dot(g1h, gt_w, preferred_element_type=f32)
         + jnp.dot(o1h, ot_w, preferred_element_type=f32)
         + jnp.dot(genres[...], genre_w, preferred_element_type=f32))
    h = jnp.maximum(h, 0.0)
    out[...] = jnp.dot(h, w2[...], preferred_element_type=f32) + b2[...]


def _mlp_call(upack, mpack, user2d, movie2d, gender2d, occ2d, genres,
              gtab, otab, wg, bg2d, w1u_lo, w1u_hi, w1m_lo, w1m_hi,
              w1, b12d, w2, b22d):
    grid = (_B // _TB,)
    full = lambda i: (0, 0)
    return pl.pallas_call(
        _mlp_body,
        grid=grid,
        in_specs=[
            pl.BlockSpec((_TB, 128), lambda i: (i, 0)),
            pl.BlockSpec((_TB, 128), lambda i: (i, 0)),
            pl.BlockSpec((_TB, 1), lambda i: (i, 0)),
            pl.BlockSpec((_TB, 1), lambda i: (i, 0)),
            pl.BlockSpec((_TB, 1), lambda i: (i, 0)),
            pl.BlockSpec((_TB, 1), lambda i: (i, 0)),
            pl.BlockSpec((_TB, _NUM_GENRES), lambda i: (i, 0)),
            pl.BlockSpec((_NUM_GENDERS, _ED), full),
            pl.BlockSpec((_NUM_OCC, _ED), full),
            pl.BlockSpec((_NUM_GENRES, _ED), full),
            pl.BlockSpec((1, _ED), full),
            pl.BlockSpec((128, _H), full),
            pl.BlockSpec((128, _H), full),
            pl.BlockSpec((128, _H), full),
            pl.BlockSpec((128, _H), full),
            pl.BlockSpec((5 * _ED, _H), full),
            pl.BlockSpec((1, _H), full),
            pl.BlockSpec((_H, 1), full),
            pl.BlockSpec((1, 1), full),
        ],
        out_specs=pl.BlockSpec((_TB, 1), lambda i: (i, 0)),
        out_shape=jax.ShapeDtypeStruct((_B, 1), jnp.float32),
    )(upack, mpack, user2d, movie2d, gender2d, occ2d, genres,
      gtab, otab, wg, bg2d, w1u_lo, w1u_hi, w1m_lo, w1m_hi,
      w1, b12d, w2, b22d)


def _pack_table(table):
    pairs = table.astype(jnp.bfloat16).reshape(table.shape[0], _ED // 2, 2)
    packed = lax.bitcast_convert_type(pairs, jnp.int32)
    return packed.reshape(-1, 128)


def _half_weights(w1_slice):
    k = jnp.arange(128)
    return w1_slice[2 * (k % 16)], w1_slice[2 * (k % 16) + 1]


def kernel(user, movie, gender, occupation, genres,
           user_table, movie_table, gender_table, occupation_table,
           W_genre, b_genre, W1, b1, W2, b2):
    user = user.astype(jnp.int32)
    movie = movie.astype(jnp.int32)
    upack, mpack = _sc_gather()(
        user.reshape(_B // _CH, _CH), movie.reshape(_B // _CH, _CH),
        _pack_table(user_table), _pack_table(movie_table))
    w1u_lo, w1u_hi = _half_weights(W1[0:32])
    w1m_lo, w1m_hi = _half_weights(W1[32:64])
    out = _mlp_call(
        upack, mpack,
        user.reshape(_B, 1), movie.reshape(_B, 1),
        gender.astype(jnp.int32).reshape(_B, 1),
        occupation.astype(jnp.int32).reshape(_B, 1),
        genres.astype(jnp.float32),
        gender_table, occupation_table,
        W_genre, b_genre.reshape(1, _ED),
        w1u_lo, w1u_hi, w1m_lo, w1m_hi,
        W1, b1.reshape(1, _H), W2, b2.reshape(1, 1),
    )
    return out.reshape(_B)


# speedup vs baseline: 1.1928x; 1.1928x over previous
import functools

import jax
import jax.numpy as jnp
from jax import lax
from jax.experimental import pallas as pl
from jax.experimental.pallas import tpu as pltpu
from jax.experimental.pallas import tpu_sc as plsc

_B = 16384
_ED = 32
_PACK = 8
_NC = 2
_NS = 16
_NW = _NC * _NS
_BPW = _B // _NW
_CH = 128
_NCH = _BPW // _CH

_NUM_GENDERS = 2
_NUM_OCC = 21
_NUM_GENRES = 18
_H = 128

_TB = 2048
_RB = 8000


def _pack_body(tab, out):
    bits = lax.bitcast_convert_type(tab[...].astype(jnp.bfloat16), jnp.int16)
    pairs = bits.reshape(_RB // 2, 2, _ED)
    lo = pairs[:, 0, :].astype(jnp.int32) & 0xFFFF
    hi = pairs[:, 1, :].astype(jnp.int32) << 16
    w = hi | lo
    q = _RB // 8
    out[...] = jnp.concatenate([w[j * q:(j + 1) * q] for j in range(4)],
                               axis=1)


def _pack_call(table):
    n = table.shape[0]
    if n % _RB:
        table = jnp.pad(table, ((0, _RB - n % _RB), (0, 0)))
        n = table.shape[0]
    return pl.pallas_call(
        _pack_body,
        grid=(n // _RB,),
        in_specs=[pl.BlockSpec((_RB, _ED), lambda i: (i, 0))],
        out_specs=pl.BlockSpec((_RB // 8, 4 * _ED), lambda i: (i, 0)),
        out_shape=jax.ShapeDtypeStruct((n // 8, 4 * _ED), jnp.int32),
    )(table)


def _sc_gather_body(uidx_hbm, midx_hbm, utab_hbm, mtab_hbm,
                    uemb_hbm, memb_hbm,
                    uidx_v, midx_v, urows_v, mrows_v, sem_u, sem_m):
    wid = lax.axis_index("s") * _NC + lax.axis_index("c")
    base = wid * _NCH
    pltpu.sync_copy(uidx_hbm.at[pl.ds(base, _NCH)], uidx_v)
    pltpu.sync_copy(midx_hbm.at[pl.ds(base, _NCH)], midx_v)
    for c in range(_NCH):
        cu = pltpu.async_copy(utab_hbm.at[uidx_v.at[c]], urows_v, sem_u)
        cm = pltpu.async_copy(mtab_hbm.at[midx_v.at[c]], mrows_v, sem_m)
        cu.wait()
        cm.wait()
        row0 = (base + c) * _CH
        pltpu.sync_copy(urows_v, uemb_hbm.at[pl.ds(row0, _CH)])
        pltpu.sync_copy(mrows_v, memb_hbm.at[pl.ds(row0, _CH)])


@functools.cache
def _sc_gather():
    return pl.kernel(
        _sc_gather_body,
        out_type=(jax.ShapeDtypeStruct((_B, 128), jnp.int32),
                  jax.ShapeDtypeStruct((_B, 128), jnp.int32)),
        mesh=plsc.VectorSubcoreMesh(core_axis_name="c", subcore_axis_name="s",
                                    num_cores=_NC, num_subcores=_NS),
        scratch_types=[
            pltpu.VMEM((_NCH, _CH), jnp.int32),
            pltpu.VMEM((_NCH, _CH), jnp.int32),
            pltpu.VMEM((_CH, 128), jnp.int32),
            pltpu.VMEM((_CH, 128), jnp.int32),
            pltpu.SemaphoreType.DMA,
            pltpu.SemaphoreType.DMA,
        ],
    )


def _unpack_selected(pack, sub):
    word_grp = lax.broadcasted_iota(jnp.int32, (_TB, 128), 1) >> 5
    mask = word_grp == (sub >> 1)
    even = (sub & 1) == 0
    w_even = jnp.where(mask & even, pack, 0)
    w_odd = jnp.where(mask & (~even), pack, 0)
    lo = lax.bitcast_convert_type(
        jnp.left_shift(w_even & 0xFFFF, 16), jnp.float32)
    hi = lax.bitcast_convert_type(w_odd & jnp.int32(-65536), jnp.float32)
    return lo + hi


def _mlp_body(upack, mpack, user, movie, gender, occ, genres, gtab, otab,
              wg, bg, w1u4, w1m4, w1, b1, w2, b2, out):
    w1r = w1[...]
    f32 = jnp.float32
    genre_w = jnp.dot(wg[...], w1r[128:160, :], preferred_element_type=f32)
    gt_w = jnp.dot(gtab[...], w1r[64:96, :], preferred_element_type=f32)
    ot_w = jnp.dot(otab[...], w1r[96:128, :], preferred_element_type=f32)
    bias = b1[...] + jnp.dot(bg[...], w1r[128:160, :], preferred_element_type=f32)

    uval = _unpack_selected(upack[...], user[...])
    mval = _unpack_selected(mpack[...], movie[...])

    g1h = (lax.broadcasted_iota(jnp.int32, (_TB, _NUM_GENDERS), 1)
           == gender[...]).astype(f32)
    o1h = (lax.broadcasted_iota(jnp.int32, (_TB, _NUM_OCC), 1)
           == occ[...]).astype(f32)

    h = (bias
         + jnp.dot(uval, w1u4[...], preferred_element_type=f32)
         + jnp.dot(mval, w1m4[...], preferred_element_type=f32)
         + jnp.dot(g1h, gt_w, preferred_element_type=f32)
         + jnp.dot(o1h, ot_w, preferred_element_type=f32)
         + jnp.dot(genres[...], genre_w, preferred_element_type=f32))
    h = jnp.maximum(h, 0.0)
    out[...] = jnp.dot(h, w2[...], preferred_element_type=f32) + b2[...]


def _mlp_call(upack, mpack, user2d, movie2d, gender2d, occ2d, genres,
              gtab, otab, wg, bg2d, w1u4, w1m4, w1, b12d, w2, b22d):
    grid = (_B // _TB,)
    full = lambda i: (0, 0)
    return pl.pallas_call(
        _mlp_body,
        grid=grid,
        in_specs=[
            pl.BlockSpec((_TB, 128), lambda i: (i, 0)),
            pl.BlockSpec((_TB, 128), lambda i: (i, 0)),
            pl.BlockSpec((_TB, 1), lambda i: (i, 0)),
            pl.BlockSpec((_TB, 1), lambda i: (i, 0)),
            pl.BlockSpec((_TB, 1), lambda i: (i, 0)),
            pl.BlockSpec((_TB, 1), lambda i: (i, 0)),
            pl.BlockSpec((_TB, _NUM_GENRES), lambda i: (i, 0)),
            pl.BlockSpec((_NUM_GENDERS, _ED), full),
            pl.BlockSpec((_NUM_OCC, _ED), full),
            pl.BlockSpec((_NUM_GENRES, _ED), full),
            pl.BlockSpec((1, _ED), full),
            pl.BlockSpec((128, _H), full),
            pl.BlockSpec((128, _H), full),
            pl.BlockSpec((5 * _ED, _H), full),
            pl.BlockSpec((1, _H), full),
            pl.BlockSpec((_H, 1), full),
            pl.BlockSpec((1, 1), full),
        ],
        out_specs=pl.BlockSpec((_TB, 1), lambda i: (i, 0)),
        out_shape=jax.ShapeDtypeStruct((_B, 1), jnp.float32),
    )(upack, mpack, user2d, movie2d, gender2d, occ2d, genres,
      gtab, otab, wg, bg2d, w1u4, w1m4, w1, b12d, w2, b22d)


def _split_index(r):
    t = r >> 1
    q = _RB // 8
    row = (t // (_RB // 2)) * q + t % q
    sub = ((t // q) & 3) * 2 + (r & 1)
    return row, sub


def kernel(user, movie, gender, occupation, genres,
           user_table, movie_table, gender_table, occupation_table,
           W_genre, b_genre, W1, b1, W2, b2):
    user = user.astype(jnp.int32)
    movie = movie.astype(jnp.int32)
    urow, usub = _split_index(user)
    mrow, msub = _split_index(movie)
    upack, mpack = _sc_gather()(
        urow.reshape(_B // _CH, _CH), mrow.reshape(_B // _CH, _CH),
        _pack_call(user_table), _pack_call(movie_table))
    w1u4 = jnp.concatenate([W1[0:32]] * 4, axis=0)
    w1m4 = jnp.concatenate([W1[32:64]] * 4, axis=0)
    out = _mlp_call(
        upack, mpack,
        usub.reshape(_B, 1), msub.reshape(_B, 1),
        gender.astype(jnp.int32).reshape(_B, 1),
        occupation.astype(jnp.int32).reshape(_B, 1),
        genres.astype(jnp.float32),
        gender_table, occupation_table,
        W_genre, b_genre.reshape(1, _ED),
        w1u4, w1m4,
        W1, b1.reshape(1, _H), W2, b2.reshape(1, 1),
    )
    return out.reshape(_B)


# speedup vs baseline: 2.1525x; 1.8046x over previous
import functools

import jax
import jax.numpy as jnp
from jax import lax
from jax.experimental import pallas as pl
from jax.experimental.pallas import tpu as pltpu
from jax.experimental.pallas import tpu_sc as plsc

_B = 16384
_ED = 32
_NC = 2
_NS = 16
_NW = _NC * _NS
_BPW = _B // _NW

_NUM_GENDERS = 2
_NUM_OCC = 21
_NUM_GENRES = 18
_H = 128

_TB = 2048


def _sc_gather_body(user_hbm, movie_hbm, utab_hbm, mtab_hbm,
                    uemb_hbm, memb_hbm,
                    uidx_v, midx_v, urows_v, mrows_v, sem_u, sem_m):
    wid = lax.axis_index("s") * _NC + lax.axis_index("c")
    base = wid * _BPW
    pltpu.sync_copy(user_hbm.at[pl.ds(base, _BPW)], uidx_v)
    pltpu.sync_copy(movie_hbm.at[pl.ds(base, _BPW)], midx_v)
    cu = pltpu.async_copy(utab_hbm.at[uidx_v], urows_v, sem_u)
    cm = pltpu.async_copy(mtab_hbm.at[midx_v], mrows_v, sem_m)
    cu.wait()
    cm.wait()
    pltpu.sync_copy(urows_v, uemb_hbm.at[pl.ds(base, _BPW)])
    pltpu.sync_copy(mrows_v, memb_hbm.at[pl.ds(base, _BPW)])


@functools.cache
def _sc_gather():
    return pl.kernel(
        _sc_gather_body,
        out_type=(jax.ShapeDtypeStruct((_B, _ED), jnp.float32),
                  jax.ShapeDtypeStruct((_B, _ED), jnp.float32)),
        mesh=plsc.VectorSubcoreMesh(core_axis_name="c", subcore_axis_name="s",
                                    num_cores=_NC, num_subcores=_NS),
        scratch_types=[
            pltpu.VMEM((_BPW,), jnp.int32),
            pltpu.VMEM((_BPW,), jnp.int32),
            pltpu.VMEM((_BPW, _ED), jnp.float32),
            pltpu.VMEM((_BPW, _ED), jnp.float32),
            pltpu.SemaphoreType.DMA,
            pltpu.SemaphoreType.DMA,
        ],
        compiler_params=pltpu.CompilerParams(use_tc_tiling_on_sc=False,
                                             needs_layout_passes=False),
    )


def _mlp_body(uemb, memb, gender, occ, genres, gtab, otab,
              wg, bg, w1, b1, w2, b2, out):
    w1r = w1[...]
    f32 = jnp.float32
    genre_w = jnp.dot(wg[...], w1r[128:160, :], preferred_element_type=f32)
    gt_w = jnp.dot(gtab[...], w1r[64:96, :], preferred_element_type=f32)
    ot_w = jnp.dot(otab[...], w1r[96:128, :], preferred_element_type=f32)
    bias = b1[...] + jnp.dot(bg[...], w1r[128:160, :], preferred_element_type=f32)

    g1h = (lax.broadcasted_iota(jnp.int32, (_TB, _NUM_GENDERS), 1)
           == gender[...]).astype(f32)
    o1h = (lax.broadcasted_iota(jnp.int32, (_TB, _NUM_OCC), 1)
           == occ[...]).astype(f32)

    h = (bias
         + jnp.dot(uemb[...], w1r[0:32, :], preferred_element_type=f32)
         + jnp.dot(memb[...], w1r[32:64, :], preferred_element_type=f32)
         + jnp.dot(g1h, gt_w, preferred_element_type=f32)
         + jnp.dot(o1h, ot_w, preferred_element_type=f32)
         + jnp.dot(genres[...], genre_w, preferred_element_type=f32))
    h = jnp.maximum(h, 0.0)
    out[...] = jnp.dot(h, w2[...], preferred_element_type=f32) + b2[...]


def _mlp_call(uemb, memb, gender2d, occ2d, genres, gtab, otab,
              wg, bg2d, w1, b12d, w2, b22d):
    grid = (_B // _TB,)
    full = lambda i: (0, 0)
    return pl.pallas_call(
        _mlp_body,
        grid=grid,
        in_specs=[
            pl.BlockSpec((_TB, _ED), lambda i: (i, 0)),
            pl.BlockSpec((_TB, _ED), lambda i: (i, 0)),
            pl.BlockSpec((_TB, 1), lambda i: (i, 0)),
            pl.BlockSpec((_TB, 1), lambda i: (i, 0)),
            pl.BlockSpec((_TB, _NUM_GENRES), lambda i: (i, 0)),
            pl.BlockSpec((_NUM_GENDERS, _ED), full),
            pl.BlockSpec((_NUM_OCC, _ED), full),
            pl.BlockSpec((_NUM_GENRES, _ED), full),
            pl.BlockSpec((1, _ED), full),
            pl.BlockSpec((5 * _ED, _H), full),
            pl.BlockSpec((1, _H), full),
            pl.BlockSpec((_H, 1), full),
            pl.BlockSpec((1, 1), full),
        ],
        out_specs=pl.BlockSpec((_TB, 1), lambda i: (i, 0)),
        out_shape=jax.ShapeDtypeStruct((_B, 1), jnp.float32),
    )(uemb, memb, gender2d, occ2d, genres, gtab, otab,
      wg, bg2d, w1, b12d, w2, b22d)


def kernel(user, movie, gender, occupation, genres,
           user_table, movie_table, gender_table, occupation_table,
           W_genre, b_genre, W1, b1, W2, b2):
    user = user.astype(jnp.int32)
    movie = movie.astype(jnp.int32)
    uemb, memb = _sc_gather()(user, movie, user_table, movie_table)
    out = _mlp_call(
        uemb, memb,
        gender.astype(jnp.int32).reshape(_B, 1),
        occupation.astype(jnp.int32).reshape(_B, 1),
        genres.astype(jnp.float32),
        gender_table, occupation_table,
        W_genre, b_genre.reshape(1, _ED),
        W1, b1.reshape(1, _H), W2, b2.reshape(1, 1),
    )
    return out.reshape(_B)
